# baseline (speedup 1.0000x reference)
import functools

import jax
import jax.numpy as jnp
from jax import lax
from jax.experimental import pallas as pl
from jax.experimental.pallas import tpu as pltpu
from jax.experimental.pallas import tpu_sc as plsc

N = 10000
E = 320000
DH = 128
DC = 3
DE = 16
XW = 16
TW = 144
NC = 2
NS = 16
NW = NC * NS
EW = E // NW
KB = 200
SKB = 40
NZ = N // NS
ZCH = 25
BE = 2000
BN = 2000

_f32 = jnp.float32
_bf16 = jnp.bfloat16


def _sc_gather(th, tx, src, dst):
    mesh = plsc.VectorSubcoreMesh(core_axis_name="c", subcore_axis_name="s")
    nblk = EW // KB
    half = nblk // 2

    @functools.partial(
        pl.kernel,
        out_type=(jax.ShapeDtypeStruct((E, DH), _bf16),
                  jax.ShapeDtypeStruct((E, DH), _bf16),
                  jax.ShapeDtypeStruct((E, XW), _f32),
                  jax.ShapeDtypeStruct((E, XW), _f32)),
        mesh=mesh,
        compiler_params=pltpu.CompilerParams(use_tc_tiling_on_sc=False),
        scratch_types=(
            [pltpu.VMEM((KB,), jnp.int32) for _ in range(4)]
            + [pltpu.VMEM((KB, DH), _bf16) for _ in range(4)]
            + [pltpu.VMEM((KB, XW), _f32) for _ in range(4)]
            + [pltpu.SemaphoreType.DMA for _ in range(4)]
        ),
    )
    def k(th_hbm, tx_hbm, src_hbm, dst_hbm,
          gsh_hbm, gdh_hbm, gsx_hbm, gdx_hbm,
          si0, di0, si1, di1,
          hs0, hd0, hs1, hd1,
          xs0, xd0, xs1, xd1,
          semg0, semg1, sems0, sems1):
        wid = lax.axis_index("s") * NC + lax.axis_index("c")
        base = wid * EW
        bufs = ((si0, di0, hs0, hd0, xs0, xd0, semg0, sems0),
                (si1, di1, hs1, hd1, xs1, xd1, semg1, sems1))

        def issue(i, b):
            si, di, hs, hd, xs, xd, semg, _ = bufs[b]
            off = base + i * KB
            pltpu.sync_copy(src_hbm.at[pl.ds(off, KB)], si)
            pltpu.sync_copy(dst_hbm.at[pl.ds(off, KB)], di)
            pltpu.async_copy(th_hbm.at[si], hs, semg)
            pltpu.async_copy(th_hbm.at[di], hd, semg)
            pltpu.async_copy(tx_hbm.at[si], xs, semg)
            pltpu.async_copy(tx_hbm.at[di], xd, semg)

        def finish(i, b):
            si, di, hs, hd, xs, xd, semg, sems = bufs[b]
            off = base + i * KB
            pltpu.make_async_copy(th_hbm.at[si], hs, semg).wait()
            pltpu.make_async_copy(th_hbm.at[di], hd, semg).wait()
            pltpu.make_async_copy(tx_hbm.at[si], xs, semg).wait()
            pltpu.make_async_copy(tx_hbm.at[di], xd, semg).wait()
            pltpu.async_copy(hs, gsh_hbm.at[pl.ds(off, KB)], sems)
            pltpu.async_copy(hd, gdh_hbm.at[pl.ds(off, KB)], sems)
            pltpu.async_copy(xs, gsx_hbm.at[pl.ds(off, KB)], sems)
            pltpu.async_copy(xd, gdx_hbm.at[pl.ds(off, KB)], sems)

        def wait_stores(i, b):
            si, di, hs, hd, xs, xd, _, sems = bufs[b]
            off = base + i * KB
            pltpu.make_async_copy(hs, gsh_hbm.at[pl.ds(off, KB)], sems).wait()
            pltpu.make_async_copy(hd, gdh_hbm.at[pl.ds(off, KB)], sems).wait()
            pltpu.make_async_copy(xs, gsx_hbm.at[pl.ds(off, KB)], sems).wait()
            pltpu.make_async_copy(xd, gdx_hbm.at[pl.ds(off, KB)], sems).wait()

        issue(0, 0)

        @pl.loop(0, half)
        def _(kk):
            i0 = 2 * kk
            issue(i0 + 1, 1)
            finish(i0, 0)

            @pl.when(kk < half - 1)
            def _():
                wait_stores(i0, 0)
                issue(i0 + 2, 0)

            finish(i0 + 1, 1)
            wait_stores(i0 + 1, 1)

        wait_stores(nblk - 2, 0)

    return k(th, tx, src, dst)


def _sc_scatter(msg, dst):
    mesh = plsc.VectorSubcoreMesh(core_axis_name="c", subcore_axis_name="s")
    nblk = EW // SKB
    half = nblk // 2

    @functools.partial(
        pl.kernel,
        out_type=jax.ShapeDtypeStruct((NC * N, TW), _f32),
        mesh=mesh,
        compiler_params=pltpu.CompilerParams(use_tc_tiling_on_sc=False),
        scratch_types=[
            pltpu.VMEM((SKB,), jnp.int32),
            pltpu.VMEM((SKB,), jnp.int32),
            pltpu.VMEM((SKB, TW), _f32),
            pltpu.VMEM((SKB, TW), _f32),
            pltpu.VMEM((ZCH, TW), _f32),
            pltpu.VMEM_SHARED((N, TW), _f32),
            pltpu.SemaphoreType.DMA,
            pltpu.SemaphoreType.DMA,
            pltpu.SemaphoreType.DMA,
            pltpu.SemaphoreType.DMA,
        ],
    )
    def k(msg_hbm, dst_hbm, out_hbm,
          di0, di1, rows0, rows1, zbuf_v, acc_sh,
          seml0, seml1, sema0, sema1):
        c = lax.axis_index("c")
        s = lax.axis_index("s")
        wid = s * NC + c
        bufs = ((di0, rows0, seml0, sema0), (di1, rows1, seml1, sema1))

        @pl.loop(0, ZCH)
        def _(i):
            @pl.loop(0, TW, step=16)
            def _(j):
                zbuf_v[i, pl.ds(j, 16)] = jnp.zeros((16,), _f32)

        @pl.loop(0, NZ, step=ZCH)
        def _(r):
            pltpu.sync_copy(zbuf_v, acc_sh.at[pl.ds(s * NZ + r, ZCH)])

        plsc.subcore_barrier()

        base = wid * EW

        def issue(i, b):
            di, rows, seml, _ = bufs[b]
            off = base + i * SKB
            pltpu.sync_copy(dst_hbm.at[pl.ds(off, SKB)], di)
            pltpu.async_copy(msg_hbm.at[pl.ds(off, SKB)], rows, seml)

        def apply(i, b):
            di, rows, seml, sema = bufs[b]
            off = base + i * SKB
            pltpu.make_async_copy(msg_hbm.at[pl.ds(off, SKB)], rows,
                                  seml).wait()
            pltpu.async_copy(rows, acc_sh.at[di], sema, add=True)

        def wait_add(i, b):
            di, rows, _, sema = bufs[b]
            pltpu.make_async_copy(rows, acc_sh.at[di], sema).wait()

        issue(0, 0)

        @pl.loop(0, half)
        def _(kk):
            i0 = 2 * kk
            issue(i0 + 1, 1)
            apply(i0, 0)

            @pl.when(kk < half - 1)
            def _():
                wait_add(i0, 0)
                issue(i0 + 2, 0)

            apply(i0 + 1, 1)
            wait_add(i0 + 1, 1)

        wait_add(nblk - 2, 0)

        plsc.subcore_barrier()
        pltpu.sync_copy(acc_sh.at[pl.ds(s * NZ, NZ)],
                        out_hbm.at[pl.ds(c * N + s * NZ, NZ)])

    return k(msg, dst)


def _edge_compute(gsh, gdh, gsx, gdx, a, we1s, we1d, we1r, we1a, be1, we2,
                  be2, wa, ba, wc1s, wc1d, wc1r, wc1a, bc1, wc2, bc2, wc3):
    def body(gsh_ref, gdh_ref, gsx_ref, gdx_ref, a_ref,
             we1s_ref, we1d_ref, we1r_ref, we1a_ref, be1_ref,
             we2_ref, be2_ref, wa_ref, ba_ref,
             wc1s_ref, wc1d_ref, wc1r_ref, wc1a_ref, bc1_ref,
             wc2_ref, bc2_ref, wc3_ref, msg_ref):
        hs = gsh_ref[...]
        hd = gdh_ref[...]
        xdiff = gsx_ref[:, :DC] - gdx_ref[:, :DC]
        radial = jnp.sqrt(jnp.sum(xdiff * xdiff, axis=1, keepdims=True))
        xdn = xdiff / (radial + 1.0)
        ab = a_ref[...].astype(_bf16)

        def pre1(ws_ref, wd_ref, wr_ref, wa2_ref, b_ref):
            p = jnp.dot(hs, ws_ref[...], preferred_element_type=_f32)
            p = p + jnp.dot(hd, wd_ref[...], preferred_element_type=_f32)
            p = p + jnp.dot(ab, wa2_ref[...], preferred_element_type=_f32)
            return p + radial * wr_ref[...] + b_ref[...]

        mh = jax.nn.silu(pre1(we1s_ref, we1d_ref, we1r_ref, we1a_ref, be1_ref))
        mh = jax.nn.silu(jnp.dot(mh.astype(_bf16), we2_ref[...],
                                 preferred_element_type=_f32) + be2_ref[...])
        att = jax.nn.sigmoid(jnp.dot(mh.astype(_bf16), wa_ref[...],
                                     preferred_element_type=_f32) + ba_ref[...])
        msg_h = att * mh
        ch = jax.nn.silu(pre1(wc1s_ref, wc1d_ref, wc1r_ref, wc1a_ref, bc1_ref))
        ch = jax.nn.silu(jnp.dot(ch.astype(_bf16), wc2_ref[...],
                                 preferred_element_type=_f32) + bc2_ref[...])
        coef = jnp.dot(ch.astype(_bf16), wc3_ref[...],
                       preferred_element_type=_f32)
        msg_x = coef * xdn
        msg_ref[...] = jnp.concatenate(
            [msg_h, msg_x, jnp.zeros((BE, TW - DH - DC), _f32)], axis=1)

    full = lambda arr: pl.BlockSpec(arr.shape, lambda i: (0,) * arr.ndim)
    return pl.pallas_call(
        body,
        grid=(E // BE,),
        in_specs=[
            pl.BlockSpec((BE, DH), lambda i: (i, 0)),
            pl.BlockSpec((BE, DH), lambda i: (i, 0)),
            pl.BlockSpec((BE, XW), lambda i: (i, 0)),
            pl.BlockSpec((BE, XW), lambda i: (i, 0)),
            pl.BlockSpec((BE, DE), lambda i: (i, 0)),
            full(we1s), full(we1d), full(we1r), full(we1a), full(be1),
            full(we2), full(be2), full(wa), full(ba),
            full(wc1s), full(wc1d), full(wc1r), full(wc1a), full(bc1),
            full(wc2), full(bc2), full(wc3),
        ],
        out_specs=pl.BlockSpec((BE, TW), lambda i: (i, 0)),
        out_shape=jax.ShapeDtypeStruct((E, TW), _f32),
    )(gsh, gdh, gsx, gdx, a, we1s, we1d, we1r, we1a, be1, we2, be2, wa, ba,
      wc1s, wc1d, wc1r, wc1a, bc1, wc2, bc2, wc3)


def _node_compute(h, x, p0, p1, wn1h, wn1n, bn1, wn2, bn2):
    def body(h_ref, x_ref, p0_ref, p1_ref,
             wn1h_ref, wn1n_ref, bn1_ref, wn2_ref, bn2_ref,
             ho_ref, xo_ref):
        hn = p0_ref[:, :DH] + p1_ref[:, :DH]
        xn = p0_ref[:, DH:DH + DC] + p1_ref[:, DH:DH + DC]
        h_b = h_ref[...]
        pre = (jnp.dot(h_b.astype(_bf16), wn1h_ref[...],
                       preferred_element_type=_f32)
               + jnp.dot(hn.astype(_bf16), wn1n_ref[...],
                         preferred_element_type=_f32)
               + bn1_ref[...])
        nh = jax.nn.silu(pre)
        nh = jnp.dot(nh.astype(_bf16), wn2_ref[...],
                     preferred_element_type=_f32) + bn2_ref[...]
        ho_ref[...] = h_b + nh
        xo_ref[...] = x_ref[...] + xn

    full = lambda arr: pl.BlockSpec(arr.shape, lambda i: (0,) * arr.ndim)
    return pl.pallas_call(
        body,
        grid=(N // BN,),
        in_specs=[
            pl.BlockSpec((BN, DH), lambda i: (i, 0)),
            pl.BlockSpec((BN, DC), lambda i: (i, 0)),
            pl.BlockSpec((BN, TW), lambda i: (i, 0)),
            pl.BlockSpec((BN, TW), lambda i: (i, 0)),
            full(wn1h), full(wn1n), full(bn1), full(wn2), full(bn2),
        ],
        out_specs=[
            pl.BlockSpec((BN, DH), lambda i: (i, 0)),
            pl.BlockSpec((BN, DC), lambda i: (i, 0)),
        ],
        out_shape=[
            jax.ShapeDtypeStruct((N, DH), _f32),
            jax.ShapeDtypeStruct((N, DC), _f32),
        ],
    )(h, x, p0, p1, wn1h, wn1n, bn1, wn2, bn2)


def kernel(h, x, a, edge_index, We1, be1, We2, be2, Wa, ba, Wn1, bn1, Wn2,
           bn2, Wc1, bc1, Wc2, bc2, Wc3):
    src = edge_index[0]
    dst = edge_index[1]
    th = h.astype(_bf16)
    tx = jnp.concatenate([x, jnp.zeros((N, XW - DC), _f32)], axis=1)

    gsh, gdh, gsx, gdx = _sc_gather(th, tx, src, dst)

    bf = lambda w: w.astype(_bf16)
    row = lambda b: b.reshape(1, -1)
    msg = _edge_compute(
        gsh, gdh, gsx, gdx, a,
        bf(We1[:DH]), bf(We1[DH:2 * DH]), We1[2 * DH:2 * DH + 1],
        bf(We1[2 * DH + 1:]), row(be1),
        bf(We2), row(be2), bf(Wa), row(ba),
        bf(Wc1[:DH]), bf(Wc1[DH:2 * DH]), Wc1[2 * DH:2 * DH + 1],
        bf(Wc1[2 * DH + 1:]), row(bc1),
        bf(Wc2), row(bc2), bf(Wc3))

    parts = _sc_scatter(msg, dst)
    p0 = parts[:N]
    p1 = parts[N:]

    h_out, x_out = _node_compute(h, x, p0, p1, bf(Wn1[:DH]), bf(Wn1[DH:]),
                                 row(bn1), bf(Wn2), row(bn2))
    return (h_out, x_out)


# speedup vs baseline: 1.4880x; 1.4880x over previous
import functools

import jax
import jax.numpy as jnp
from jax import lax
from jax.experimental import pallas as pl
from jax.experimental.pallas import tpu as pltpu
from jax.experimental.pallas import tpu_sc as plsc

N = 10000
E = 320000
DH = 128
DHH = DH // 2
DC = 3
DE = 16
XW = 16
NC = 2
NS = 16
NW = NC * NS
EW = E // NW
KB = 200
SKB = 40
NZ = N // NS
ZCH = 25
BE = 2000
BN = 2000

_f32 = jnp.float32
_bf16 = jnp.bfloat16
_i32 = jnp.int32


def _sc_gather(thp, tx, src, dst):
    mesh = plsc.VectorSubcoreMesh(core_axis_name="c", subcore_axis_name="s")
    nblk = EW // KB
    half = nblk // 2

    @functools.partial(
        pl.kernel,
        out_type=(jax.ShapeDtypeStruct((E, DH), _f32),
                  jax.ShapeDtypeStruct((E, DH), _f32)),
        mesh=mesh,
        compiler_params=pltpu.CompilerParams(use_tc_tiling_on_sc=False),
        scratch_types=(
            [pltpu.VMEM((KB,), _i32) for _ in range(4)]
            + [pltpu.VMEM((KB, DHH), _f32) for _ in range(4)]
            + [pltpu.VMEM((KB, XW), _f32) for _ in range(4)]
            + [pltpu.SemaphoreType.DMA for _ in range(4)]
        ),
    )
    def k(thp_hbm, tx_hbm, src_hbm, dst_hbm, ghh_hbm, gx_hbm,
          si0, di0, si1, di1,
          hs0, hd0, hs1, hd1,
          xs0, xd0, xs1, xd1,
          semg0, semg1, sems0, sems1):
        wid = lax.axis_index("s") * NC + lax.axis_index("c")
        base = wid * EW
        bufs = ((si0, di0, hs0, hd0, xs0, xd0, semg0, sems0),
                (si1, di1, hs1, hd1, xs1, xd1, semg1, sems1))

        def issue(i, b):
            si, di, hs, hd, xs, xd, semg, _ = bufs[b]
            off = base + i * KB
            pltpu.sync_copy(src_hbm.at[pl.ds(off, KB)], si)
            pltpu.sync_copy(dst_hbm.at[pl.ds(off, KB)], di)
            pltpu.async_copy(thp_hbm.at[si], hs, semg)
            pltpu.async_copy(thp_hbm.at[di], hd, semg)
            pltpu.async_copy(tx_hbm.at[si], xs, semg)
            pltpu.async_copy(tx_hbm.at[di], xd, semg)

        def finish(i, b):
            si, di, hs, hd, xs, xd, semg, sems = bufs[b]
            off = base + i * KB
            pltpu.make_async_copy(thp_hbm.at[si], hs, semg).wait()
            pltpu.make_async_copy(thp_hbm.at[di], hd, semg).wait()
            pltpu.make_async_copy(tx_hbm.at[si], xs, semg).wait()
            pltpu.make_async_copy(tx_hbm.at[di], xd, semg).wait()
            pltpu.async_copy(
                hs, ghh_hbm.at[pl.ds(off, KB), pl.ds(0, DHH)], sems)
            pltpu.async_copy(
                hd, ghh_hbm.at[pl.ds(off, KB), pl.ds(DHH, DHH)], sems)
            pltpu.async_copy(
                xs, gx_hbm.at[pl.ds(off, KB), pl.ds(0, XW)], sems)
            pltpu.async_copy(
                xd, gx_hbm.at[pl.ds(off, KB), pl.ds(XW, XW)], sems)

        def wait_stores(i, b):
            si, di, hs, hd, xs, xd, _, sems = bufs[b]
            off = base + i * KB
            pltpu.make_async_copy(
                hs, ghh_hbm.at[pl.ds(off, KB), pl.ds(0, DHH)], sems).wait()
            pltpu.make_async_copy(
                hd, ghh_hbm.at[pl.ds(off, KB), pl.ds(DHH, DHH)], sems).wait()
            pltpu.make_async_copy(
                xs, gx_hbm.at[pl.ds(off, KB), pl.ds(0, XW)], sems).wait()
            pltpu.make_async_copy(
                xd, gx_hbm.at[pl.ds(off, KB), pl.ds(XW, XW)], sems).wait()

        issue(0, 0)

        @pl.loop(0, half)
        def _(kk):
            i0 = 2 * kk
            issue(i0 + 1, 1)
            finish(i0, 0)

            @pl.when(kk < half - 1)
            def _():
                wait_stores(i0, 0)
                issue(i0 + 2, 0)

            finish(i0 + 1, 1)
            wait_stores(i0 + 1, 1)

        wait_stores(nblk - 2, 0)

    return k(thp, tx, src, dst)


def _sc_scatter(msg_h, msg_x, dst):
    mesh = plsc.VectorSubcoreMesh(core_axis_name="c", subcore_axis_name="s")
    nblk = EW // SKB
    half = nblk // 2

    @functools.partial(
        pl.kernel,
        out_type=(jax.ShapeDtypeStruct((NC * N, DH), _f32),
                  jax.ShapeDtypeStruct((NC * N, DH), _f32)),
        mesh=mesh,
        compiler_params=pltpu.CompilerParams(use_tc_tiling_on_sc=False),
        scratch_types=[
            pltpu.VMEM((SKB,), _i32),
            pltpu.VMEM((SKB,), _i32),
            pltpu.VMEM((SKB, DH), _f32),
            pltpu.VMEM((SKB, DH), _f32),
            pltpu.VMEM((SKB, XW), _f32),
            pltpu.VMEM((SKB, XW), _f32),
            pltpu.VMEM((ZCH, DH), _f32),
            pltpu.VMEM((ZCH, XW), _f32),
            pltpu.VMEM_SHARED((N, DH), _f32),
            pltpu.VMEM_SHARED((N, XW), _f32),
            pltpu.SemaphoreType.DMA,
            pltpu.SemaphoreType.DMA,
            pltpu.SemaphoreType.DMA,
            pltpu.SemaphoreType.DMA,
        ],
    )
    def k(msgh_hbm, msgx_hbm, dst_hbm, outh_hbm, outx_hbm,
          di0, di1, rh0, rh1, rx0, rx1, zh_v, zx_v, acch_sh, accx_sh,
          seml0, seml1, sema0, sema1):
        c = lax.axis_index("c")
        s = lax.axis_index("s")
        wid = s * NC + c
        bufs = ((di0, rh0, rx0, seml0, sema0), (di1, rh1, rx1, seml1, sema1))

        @pl.loop(0, ZCH)
        def _(i):
            @pl.loop(0, DH, step=16)
            def _(j):
                zh_v[i, pl.ds(j, 16)] = jnp.zeros((16,), _f32)
            zx_v[i, pl.ds(0, 16)] = jnp.zeros((16,), _f32)

        @pl.loop(0, NZ, step=ZCH)
        def _(r):
            pltpu.sync_copy(zh_v, acch_sh.at[pl.ds(s * NZ + r, ZCH)])
            pltpu.sync_copy(zx_v, accx_sh.at[pl.ds(s * NZ + r, ZCH)])

        plsc.subcore_barrier()

        base = wid * EW

        def issue(i, b):
            di, rh, rx, seml, _ = bufs[b]
            off = base + i * SKB
            pltpu.sync_copy(dst_hbm.at[pl.ds(off, SKB)], di)
            pltpu.async_copy(msgh_hbm.at[pl.ds(off, SKB)], rh, seml)
            pltpu.async_copy(
                msgx_hbm.at[pl.ds(off, SKB), pl.ds(0, XW)], rx, seml)

        def apply(i, b):
            di, rh, rx, seml, sema = bufs[b]
            off = base + i * SKB
            pltpu.make_async_copy(msgh_hbm.at[pl.ds(off, SKB)], rh,
                                  seml).wait()
            pltpu.make_async_copy(
                msgx_hbm.at[pl.ds(off, SKB), pl.ds(0, XW)], rx, seml).wait()
            pltpu.async_copy(rh, acch_sh.at[di], sema, add=True)
            pltpu.async_copy(rx, accx_sh.at[di], sema, add=True)

        def wait_add(b):
            di, rh, rx, _, sema = bufs[b]
            pltpu.make_async_copy(rh, acch_sh.at[di], sema).wait()
            pltpu.make_async_copy(rx, accx_sh.at[di], sema).wait()

        issue(0, 0)

        @pl.loop(0, half)
        def _(kk):
            i0 = 2 * kk
            issue(i0 + 1, 1)
            apply(i0, 0)

            @pl.when(kk < half - 1)
            def _():
                wait_add(0)
                issue(i0 + 2, 0)

            apply(i0 + 1, 1)
            wait_add(1)

        wait_add(0)

        plsc.subcore_barrier()
        pltpu.sync_copy(acch_sh.at[pl.ds(s * NZ, NZ)],
                        outh_hbm.at[pl.ds(c * N + s * NZ, NZ)])
        pltpu.sync_copy(accx_sh.at[pl.ds(s * NZ, NZ)],
                        outx_hbm.at[pl.ds(c * N + s * NZ, NZ), pl.ds(0, XW)])

    return k(msg_h, msg_x, dst)


def _unpack_pair(words_f32):
    w = lax.bitcast_convert_type(words_f32, _i32)
    lo = lax.bitcast_convert_type(w << 16, _f32)
    hi = lax.bitcast_convert_type((w >> 16) << 16, _f32)
    return lo.astype(_bf16), hi.astype(_bf16)


def _edge_compute(ghh, gx, a, we1s_lo, we1s_hi, we1d_lo, we1d_hi, we1r,
                  we1a, be1, we2, be2, wa, ba, wc1s_lo, wc1s_hi, wc1d_lo,
                  wc1d_hi, wc1r, wc1a, bc1, wc2, bc2, wc3):
    def body(ghh_ref, gx_ref, a_ref,
             we1s_lo_ref, we1s_hi_ref, we1d_lo_ref, we1d_hi_ref,
             we1r_ref, we1a_ref, be1_ref,
             we2_ref, be2_ref, wa_ref, ba_ref,
             wc1s_lo_ref, wc1s_hi_ref, wc1d_lo_ref, wc1d_hi_ref,
             wc1r_ref, wc1a_ref, bc1_ref,
             wc2_ref, bc2_ref, wc3_ref, msgh_ref, msgx_ref):
        hs_lo, hs_hi = _unpack_pair(ghh_ref[:, :DHH])
        hd_lo, hd_hi = _unpack_pair(ghh_ref[:, DHH:])
        gx_b = gx_ref[...]
        xdiff = gx_b[:, :DC] - gx_b[:, XW:XW + DC]
        radial = jnp.sqrt(jnp.sum(xdiff * xdiff, axis=1, keepdims=True))
        xdn = xdiff / (radial + 1.0)
        ab = a_ref[...].astype(_bf16)

        def pre1(wsl, wsh, wdl, wdh, wr, wa2, b):
            p = jnp.dot(hs_lo, wsl[...], preferred_element_type=_f32)
            p = p + jnp.dot(hs_hi, wsh[...], preferred_element_type=_f32)
            p = p + jnp.dot(hd_lo, wdl[...], preferred_element_type=_f32)
            p = p + jnp.dot(hd_hi, wdh[...], preferred_element_type=_f32)
            p = p + jnp.dot(ab, wa2[...], preferred_element_type=_f32)
            return p + radial * wr[...] + b[...]

        mh = jax.nn.silu(pre1(we1s_lo_ref, we1s_hi_ref, we1d_lo_ref,
                              we1d_hi_ref, we1r_ref, we1a_ref, be1_ref))
        mh = jax.nn.silu(jnp.dot(mh.astype(_bf16), we2_ref[...],
                                 preferred_element_type=_f32) + be2_ref[...])
        att = jax.nn.sigmoid(jnp.dot(mh.astype(_bf16), wa_ref[...],
                                     preferred_element_type=_f32) + ba_ref[...])
        msgh_ref[...] = att * mh
        ch = jax.nn.silu(pre1(wc1s_lo_ref, wc1s_hi_ref, wc1d_lo_ref,
                              wc1d_hi_ref, wc1r_ref, wc1a_ref, bc1_ref))
        ch = jax.nn.silu(jnp.dot(ch.astype(_bf16), wc2_ref[...],
                                 preferred_element_type=_f32) + bc2_ref[...])
        coef = jnp.dot(ch.astype(_bf16), wc3_ref[...],
                       preferred_element_type=_f32)
        msgx_ref[...] = jnp.concatenate(
            [coef * xdn, jnp.zeros((BE, DH - DC), _f32)], axis=1)

    full = lambda arr: pl.BlockSpec(arr.shape, lambda i: (0,) * arr.ndim)
    return pl.pallas_call(
        body,
        grid=(E // BE,),
        in_specs=[
            pl.BlockSpec((BE, DH), lambda i: (i, 0)),
            pl.BlockSpec((BE, DH), lambda i: (i, 0)),
            pl.BlockSpec((BE, DE), lambda i: (i, 0)),
            full(we1s_lo), full(we1s_hi), full(we1d_lo), full(we1d_hi),
            full(we1r), full(we1a), full(be1),
            full(we2), full(be2), full(wa), full(ba),
            full(wc1s_lo), full(wc1s_hi), full(wc1d_lo), full(wc1d_hi),
            full(wc1r), full(wc1a), full(bc1),
            full(wc2), full(bc2), full(wc3),
        ],
        out_specs=[
            pl.BlockSpec((BE, DH), lambda i: (i, 0)),
            pl.BlockSpec((BE, DH), lambda i: (i, 0)),
        ],
        out_shape=[
            jax.ShapeDtypeStruct((E, DH), _f32),
            jax.ShapeDtypeStruct((E, DH), _f32),
        ],
    )(ghh, gx, a, we1s_lo, we1s_hi, we1d_lo, we1d_hi, we1r, we1a, be1,
      we2, be2, wa, ba, wc1s_lo, wc1s_hi, wc1d_lo, wc1d_hi, wc1r, wc1a,
      bc1, wc2, bc2, wc3)


def _node_compute(h, x, ph0, ph1, px0, px1, wn1h, wn1n, bn1, wn2, bn2):
    def body(h_ref, x_ref, ph0_ref, ph1_ref, px0_ref, px1_ref,
             wn1h_ref, wn1n_ref, bn1_ref, wn2_ref, bn2_ref,
             ho_ref, xo_ref):
        hn = ph0_ref[...] + ph1_ref[...]
        xn = px0_ref[:, :DC] + px1_ref[:, :DC]
        h_b = h_ref[...]
        pre = (jnp.dot(h_b.astype(_bf16), wn1h_ref[...],
                       preferred_element_type=_f32)
               + jnp.dot(hn.astype(_bf16), wn1n_ref[...],
                         preferred_element_type=_f32)
               + bn1_ref[...])
        nh = jax.nn.silu(pre)
        nh = jnp.dot(nh.astype(_bf16), wn2_ref[...],
                     preferred_element_type=_f32) + bn2_ref[...]
        ho_ref[...] = h_b + nh
        xo_ref[...] = x_ref[...] + xn

    full = lambda arr: pl.BlockSpec(arr.shape, lambda i: (0,) * arr.ndim)
    return pl.pallas_call(
        body,
        grid=(N // BN,),
        in_specs=[
            pl.BlockSpec((BN, DH), lambda i: (i, 0)),
            pl.BlockSpec((BN, DC), lambda i: (i, 0)),
            pl.BlockSpec((BN, DH), lambda i: (i, 0)),
            pl.BlockSpec((BN, DH), lambda i: (i, 0)),
            pl.BlockSpec((BN, DH), lambda i: (i, 0)),
            pl.BlockSpec((BN, DH), lambda i: (i, 0)),
            full(wn1h), full(wn1n), full(bn1), full(wn2), full(bn2),
        ],
        out_specs=[
            pl.BlockSpec((BN, DH), lambda i: (i, 0)),
            pl.BlockSpec((BN, DC), lambda i: (i, 0)),
        ],
        out_shape=[
            jax.ShapeDtypeStruct((N, DH), _f32),
            jax.ShapeDtypeStruct((N, DC), _f32),
        ],
    )(h, x, ph0, ph1, px0, px1, wn1h, wn1n, bn1, wn2, bn2)


def kernel(h, x, a, edge_index, We1, be1, We2, be2, Wa, ba, Wn1, bn1, Wn2,
           bn2, Wc1, bc1, Wc2, bc2, Wc3):
    src = edge_index[0]
    dst = edge_index[1]
    lo = lax.bitcast_convert_type(
        h[:, :DHH].astype(_bf16), jnp.uint16).astype(jnp.uint32)
    hi = lax.bitcast_convert_type(
        h[:, DHH:].astype(_bf16), jnp.uint16).astype(jnp.uint32)
    thp = lax.bitcast_convert_type((hi << 16) | lo, _f32)
    tx = jnp.concatenate([x, jnp.zeros((N, XW - DC), _f32)], axis=1)

    ghh, gx = _sc_gather(thp, tx, src, dst)

    bf = lambda w: w.astype(_bf16)
    row = lambda b: b.reshape(1, -1)
    msg_h, msg_x = _edge_compute(
        ghh, gx, a,
        bf(We1[:DHH]), bf(We1[DHH:DH]), bf(We1[DH:DH + DHH]),
        bf(We1[DH + DHH:2 * DH]), We1[2 * DH:2 * DH + 1],
        bf(We1[2 * DH + 1:]), row(be1),
        bf(We2), row(be2), bf(Wa), row(ba),
        bf(Wc1[:DHH]), bf(Wc1[DHH:DH]), bf(Wc1[DH:DH + DHH]),
        bf(Wc1[DH + DHH:2 * DH]), Wc1[2 * DH:2 * DH + 1],
        bf(Wc1[2 * DH + 1:]), row(bc1),
        bf(Wc2), row(bc2), bf(Wc3))

    parts_h, parts_x = _sc_scatter(msg_h, msg_x, dst)

    h_out, x_out = _node_compute(
        h, x, parts_h[:N], parts_h[N:], parts_x[:N], parts_x[N:],
        bf(Wn1[:DH]), bf(Wn1[DH:]), row(bn1), bf(Wn2), row(bn2))
    return (h_out, x_out)
